# baseline (speedup 1.0000x reference)
import functools

import jax
import jax.numpy as jnp
from jax import lax
from jax.experimental import pallas as pl
from jax.experimental.pallas import tpu as pltpu
from jax.experimental.pallas import tpu_sc as plsc

N = 10000
E = 320000
D = 128
G = 64

NC = 2
NS = 16
NW = NC * NS
EPW = E // NW
CH = 80
NCHUNK = EPW // CH
CH_S = 112
NCHUNK_S = 92
EPW_S = CH_S * NCHUNK_S
E_PAD_S = NW * EPW_S
N_ACC = 10016
ZTILES = 10
ZROWS = 1000
N_PAD = 10240
RPT = N_PAD // NS

CH_D = 128
NCHUNK_D = 80
EPW_D = CH_D * NCHUNK_D
E_PAD = NW * EPW_D

BN = 1000
NBLK = N // BN

_sc_mesh = plsc.VectorSubcoreMesh(core_axis_name="core", subcore_axis_name="subcore")


def _deg_kernel(dstp, ones_rows, zeros16):
    @functools.partial(
        pl.kernel,
        out_type=jax.ShapeDtypeStruct((NC, N_PAD, 16), jnp.float32),
        mesh=_sc_mesh,
        scratch_types=[
            pltpu.VMEM_SHARED((N_PAD, 16), jnp.float32),
            pltpu.VMEM((NCHUNK, 1, CH), jnp.int32),
            pltpu.VMEM((CH, 16), jnp.float32),
        ],
    )
    def k(dst_hbm, ones_hbm, zeros_hbm, out_hbm, acc, idx_v, ones_v):
        cid = lax.axis_index("core")
        sid = lax.axis_index("subcore")
        wid = cid * NS + sid
        pltpu.sync_copy(zeros_hbm, acc.at[pl.ds(sid * RPT, RPT)])
        pltpu.sync_copy(ones_hbm, ones_v)
        pltpu.sync_copy(dst_hbm.at[wid], idx_v)
        plsc.subcore_barrier()

        @pl.loop(0, NCHUNK)
        def _(j):
            pltpu.sync_copy(ones_v, acc.at[idx_v.at[j, 0]], add=True)

        plsc.subcore_barrier()
        pltpu.sync_copy(acc.at[pl.ds(sid * RPT, RPT)],
                        out_hbm.at[cid, pl.ds(sid * RPT, RPT)])

    return k(dstp, ones_rows, zeros16)


def _edge_scatter(S, src3, dst4, zeros128):
    @functools.partial(
        pl.kernel,
        out_type=jax.ShapeDtypeStruct((NC, N_ACC, D), jnp.float32),
        mesh=_sc_mesh,
        scratch_types=[
            pltpu.VMEM_SHARED((N_ACC, D), jnp.float32),
            pltpu.VMEM((EPW_S,), jnp.int32),
            pltpu.VMEM((NCHUNK_S, 1, CH_S), jnp.int32),
            pltpu.VMEM((2, CH_S, D), jnp.float32),
            pltpu.SemaphoreType.DMA((2,)),
        ],
    )
    def k(s_hbm, src_hbm, dst_hbm, zeros_hbm, out_hbm, acc, src_v, dst_v, rows_v,
          sems):
        cid = lax.axis_index("core")
        sid = lax.axis_index("subcore")
        wid = cid * NS + sid

        @pl.when(sid < ZTILES)
        def _():
            pltpu.sync_copy(zeros_hbm, acc.at[pl.ds(sid * ZROWS, ZROWS)])

        pltpu.sync_copy(src_hbm.at[wid, 0], src_v)
        pltpu.sync_copy(dst_hbm.at[wid], dst_v)
        plsc.subcore_barrier()

        def gather_start(j, b):
            pltpu.make_async_copy(s_hbm.at[src_v.at[pl.ds(j * CH_S, CH_S)]],
                                  rows_v.at[b], sems.at[b]).start()

        def gather_wait(j, b):
            pltpu.make_async_copy(s_hbm.at[src_v.at[pl.ds(j * CH_S, CH_S)]],
                                  rows_v.at[b], sems.at[b]).wait()

        gather_start(0, 0)

        @pl.loop(0, NCHUNK_S - 2, step=2)
        def _(j):
            for b in (0, 1):
                jj = j + b
                gather_start(jj + 1, 1 - b)
                gather_wait(jj, b)
                pltpu.sync_copy(rows_v.at[b], acc.at[dst_v.at[jj, 0]], add=True)

        gather_start(NCHUNK_S - 1, 1)
        gather_wait(NCHUNK_S - 2, 0)
        pltpu.sync_copy(rows_v.at[0], acc.at[dst_v.at[NCHUNK_S - 2, 0]], add=True)
        gather_wait(NCHUNK_S - 1, 1)
        pltpu.sync_copy(rows_v.at[1], acc.at[dst_v.at[NCHUNK_S - 1, 0]], add=True)

        plsc.subcore_barrier()

        @pl.when(sid < ZTILES)
        def _():
            pltpu.sync_copy(acc.at[pl.ds(sid * ZROWS, ZROWS)],
                            out_hbm.at[cid, pl.ds(sid * ZROWS, ZROWS)])

    return k(S, src3, dst4, zeros128)


def _dinv_from_deg(deg_blk):
    deg = deg_blk[0, :, 0] + deg_blk[1, :, 0] + 1.0
    return lax.rsqrt(deg)


def _k0_body(x_ref, w_ref, xw_ref):
    xw_ref[...] = jnp.dot(x_ref[...], w_ref[...],
                          preferred_element_type=jnp.float32)


def _k0(x, W1):
    return pl.pallas_call(
        _k0_body,
        grid=(NBLK,),
        in_specs=[
            pl.BlockSpec((BN, D), lambda i: (i, 0)),
            pl.BlockSpec((D, D), lambda i: (0, 0)),
        ],
        out_specs=pl.BlockSpec((BN, D), lambda i: (i, 0)),
        out_shape=jax.ShapeDtypeStruct((N, D), jnp.float32),
    )(x, W1)


def _k1s_body(xw_ref, deg_ref, s_ref):
    dinv = _dinv_from_deg(deg_ref)
    s_ref[...] = dinv[:, None] * xw_ref[...]


def _k1s(xw, deg_p):
    return pl.pallas_call(
        _k1s_body,
        grid=(NBLK,),
        in_specs=[
            pl.BlockSpec((BN, D), lambda i: (i, 0)),
            pl.BlockSpec((NC, BN, 16), lambda i: (0, i, 0)),
        ],
        out_specs=pl.BlockSpec((BN, D), lambda i: (i, 0)),
        out_shape=jax.ShapeDtypeStruct((N, D), jnp.float32),
    )(xw, deg_p)


def _k2_body(p_ref, s1_ref, deg_ref, b1_ref, w2_ref, s2_ref):
    dinv = _dinv_from_deg(deg_ref)
    agg = p_ref[0] + p_ref[1] + s1_ref[...]
    h1 = jnp.maximum(dinv[:, None] * agg + b1_ref[...], 0.0)
    xw = jnp.dot(h1, w2_ref[...], preferred_element_type=jnp.float32)
    s2_ref[...] = dinv[:, None] * xw


def _k2(P1, S1, deg_p, b1, W2):
    return pl.pallas_call(
        _k2_body,
        grid=(NBLK,),
        in_specs=[
            pl.BlockSpec((NC, BN, D), lambda i: (0, i, 0)),
            pl.BlockSpec((BN, D), lambda i: (i, 0)),
            pl.BlockSpec((NC, BN, 16), lambda i: (0, i, 0)),
            pl.BlockSpec((1, D), lambda i: (0, 0)),
            pl.BlockSpec((D, D), lambda i: (0, 0)),
        ],
        out_specs=pl.BlockSpec((BN, D), lambda i: (i, 0)),
        out_shape=jax.ShapeDtypeStruct((N, D), jnp.float32),
    )(P1, S1, deg_p, b1, W2)


def _k3_body(p_ref, s2_ref, deg_ref, b2_ref, batch_ref, wfc_ref, bfc_ref,
             out_ref, sums_ref, cnt_ref):
    i = pl.program_id(0)

    @pl.when(i == 0)
    def _():
        sums_ref[...] = jnp.zeros_like(sums_ref)
        cnt_ref[...] = jnp.zeros_like(cnt_ref)

    dinv = _dinv_from_deg(deg_ref)
    agg = p_ref[0] + p_ref[1] + s2_ref[...]
    h2 = dinv[:, None] * agg + b2_ref[...]

    b = batch_ref[0, 0, :]
    gid = lax.broadcasted_iota(jnp.int32, (G, BN), 0)
    M = (gid == b[None, :]).astype(jnp.float32)
    sums_ref[...] += jnp.dot(M, h2, preferred_element_type=jnp.float32)
    cnt_ref[...] += jnp.sum(M, axis=1)[:, None]

    @pl.when(i == NBLK - 1)
    def _():
        mean = sums_ref[...] / jnp.maximum(cnt_ref[...], 1.0)
        out_ref[...] = (jnp.dot(mean, wfc_ref[...],
                                preferred_element_type=jnp.float32)
                        + bfc_ref[...])


def _k3(P2, S2, deg_p, b2, batch3, Wfc, bfc):
    return pl.pallas_call(
        _k3_body,
        grid=(NBLK,),
        in_specs=[
            pl.BlockSpec((NC, BN, D), lambda i: (0, i, 0)),
            pl.BlockSpec((BN, D), lambda i: (i, 0)),
            pl.BlockSpec((NC, BN, 16), lambda i: (0, i, 0)),
            pl.BlockSpec((1, D), lambda i: (0, 0)),
            pl.BlockSpec((1, 1, BN), lambda i: (i, 0, 0)),
            pl.BlockSpec((D, D), lambda i: (0, 0)),
            pl.BlockSpec((1, D), lambda i: (0, 0)),
        ],
        out_specs=pl.BlockSpec((G, D), lambda i: (0, 0)),
        out_shape=jax.ShapeDtypeStruct((G, D), jnp.float32),
        scratch_shapes=[
            pltpu.VMEM((G, D), jnp.float32),
            pltpu.VMEM((G, D), jnp.float32),
        ],
    )(P2, S2, deg_p, b2, batch3, Wfc, bfc)


def kernel(x, edge_index, batch, W1, b1, W2, b2, Wfc, bfc):
    pad = E_PAD_S - E
    srcp = jnp.concatenate([edge_index[0], jnp.zeros((pad,), jnp.int32)])
    dstp = jnp.concatenate([edge_index[1], jnp.full((pad,), N, jnp.int32)])
    src3 = srcp.reshape(NW, 1, EPW_S)
    dst4s = dstp.reshape(NW, NCHUNK_S, 1, CH_S)
    dst4 = edge_index[1].reshape(NW, NCHUNK, 1, CH)
    batch3 = batch.reshape(NBLK, 1, BN)
    ones_rows = jnp.ones((CH, 16), jnp.float32)
    zeros16 = jnp.zeros((RPT, 16), jnp.float32)
    zeros128 = jnp.zeros((ZROWS, D), jnp.float32)

    deg_p = _deg_kernel(dst4, ones_rows, zeros16)

    XW1 = _k0(x, W1)
    S1 = _k1s(XW1, deg_p)
    P1 = _edge_scatter(S1, src3, dst4s, zeros128)
    S2 = _k2(P1, S1, deg_p, b1.reshape(1, D), W2)
    P2 = _edge_scatter(S2, src3, dst4s, zeros128)
    out = _k3(P2, S2, deg_p, b2.reshape(1, D), batch3, Wfc, bfc.reshape(1, D))
    return (out, batch)


# speedup vs baseline: 2.9705x; 2.9705x over previous
import functools

import jax
import jax.numpy as jnp
from jax import lax
from jax.experimental import pallas as pl
from jax.experimental.pallas import tpu as pltpu
from jax.experimental.pallas import tpu_sc as plsc

N = 10000
E = 320000
D = 128
G = 64

NC = 2
NS = 16
NW = NC * NS
EPW = E // NW
CH = 80
NCHUNK = EPW // CH
N_PAD = 10240
RPT = N_PAD // NS

BN = 1000
NBLK = N // BN

_sc_mesh = plsc.VectorSubcoreMesh(core_axis_name="core", subcore_axis_name="subcore")


def _deg_kernel(dstp, ones_rows, zeros16):
    @functools.partial(
        pl.kernel,
        out_type=jax.ShapeDtypeStruct((NC, N_PAD, 16), jnp.float32),
        mesh=_sc_mesh,
        scratch_types=[
            pltpu.VMEM_SHARED((N_PAD, 16), jnp.float32),
            pltpu.VMEM((NCHUNK, 1, CH), jnp.int32),
            pltpu.VMEM((CH, 16), jnp.float32),
        ],
    )
    def k(dst_hbm, ones_hbm, zeros_hbm, out_hbm, acc, idx_v, ones_v):
        cid = lax.axis_index("core")
        sid = lax.axis_index("subcore")
        wid = cid * NS + sid
        pltpu.sync_copy(zeros_hbm, acc.at[pl.ds(sid * RPT, RPT)])
        pltpu.sync_copy(ones_hbm, ones_v)
        pltpu.sync_copy(dst_hbm.at[wid], idx_v)
        plsc.subcore_barrier()

        @pl.loop(0, NCHUNK)
        def _(j):
            pltpu.sync_copy(ones_v, acc.at[idx_v.at[j, 0]], add=True)

        plsc.subcore_barrier()
        pltpu.sync_copy(acc.at[pl.ds(sid * RPT, RPT)],
                        out_hbm.at[cid, pl.ds(sid * RPT, RPT)])

    return k(dstp, ones_rows, zeros16)


def _edge_scatter(S, src3, dst4, zeros128):
    @functools.partial(
        pl.kernel,
        out_type=jax.ShapeDtypeStruct((NC, N_PAD, D), jnp.float32),
        mesh=_sc_mesh,
        scratch_types=[
            pltpu.VMEM_SHARED((N_PAD, D), jnp.float32),
            pltpu.VMEM((EPW,), jnp.int32),
            pltpu.VMEM((NCHUNK, 1, CH), jnp.int32),
            pltpu.VMEM((2, CH, D), jnp.float32),
            pltpu.SemaphoreType.DMA((2,)),
        ],
    )
    def k(s_hbm, src_hbm, dst_hbm, zeros_hbm, out_hbm, acc, src_v, dst_v, rows_v,
          sems):
        cid = lax.axis_index("core")
        sid = lax.axis_index("subcore")
        wid = cid * NS + sid
        pltpu.sync_copy(zeros_hbm, acc.at[pl.ds(sid * RPT, RPT)])
        pltpu.sync_copy(src_hbm.at[wid, 0], src_v)
        pltpu.sync_copy(dst_hbm.at[wid], dst_v)
        plsc.subcore_barrier()

        def gather_start(j, b):
            pltpu.make_async_copy(s_hbm.at[src_v.at[pl.ds(j * CH, CH)]],
                                  rows_v.at[b], sems.at[b]).start()

        def gather_wait(j, b):
            pltpu.make_async_copy(s_hbm.at[src_v.at[pl.ds(j * CH, CH)]],
                                  rows_v.at[b], sems.at[b]).wait()

        gather_start(0, 0)

        @pl.loop(0, NCHUNK - 1, step=2)
        def _(j):
            for b in (0, 1):
                jj = j + b
                gather_start(jj + 1, 1 - b)
                gather_wait(jj, b)
                pltpu.sync_copy(rows_v.at[b], acc.at[dst_v.at[jj, 0]], add=True)

        gather_wait(NCHUNK - 1, 0)
        pltpu.sync_copy(rows_v.at[0], acc.at[dst_v.at[NCHUNK - 1, 0]], add=True)

        plsc.subcore_barrier()
        pltpu.sync_copy(acc.at[pl.ds(sid * RPT, RPT)],
                        out_hbm.at[cid, pl.ds(sid * RPT, RPT)])

    return k(S, src3, dst4, zeros128)


def _dinv_from_deg(deg_blk):
    deg = deg_blk[0, :, 0] + deg_blk[1, :, 0] + 1.0
    return lax.rsqrt(deg)


def _k1_body(x_ref, w_ref, deg_ref, s_ref):
    xw = jnp.dot(x_ref[...], w_ref[...], preferred_element_type=jnp.float32)
    dinv = _dinv_from_deg(deg_ref)
    s_ref[...] = dinv[:, None] * xw


def _k1(x, W1, deg_p):
    return pl.pallas_call(
        _k1_body,
        grid=(NBLK,),
        in_specs=[
            pl.BlockSpec((BN, D), lambda i: (i, 0)),
            pl.BlockSpec((D, D), lambda i: (0, 0)),
            pl.BlockSpec((NC, BN, 16), lambda i: (0, i, 0)),
        ],
        out_specs=pl.BlockSpec((BN, D), lambda i: (i, 0)),
        out_shape=jax.ShapeDtypeStruct((N, D), jnp.float32),
    )(x, W1, deg_p)


def _k2_body(p_ref, s1_ref, deg_ref, b1_ref, w2_ref, s2_ref):
    dinv = _dinv_from_deg(deg_ref)
    agg = p_ref[0] + p_ref[1] + s1_ref[...]
    h1 = jnp.maximum(dinv[:, None] * agg + b1_ref[...], 0.0)
    xw = jnp.dot(h1, w2_ref[...], preferred_element_type=jnp.float32)
    s2_ref[...] = dinv[:, None] * xw


def _k2(P1, S1, deg_p, b1, W2):
    return pl.pallas_call(
        _k2_body,
        grid=(NBLK,),
        in_specs=[
            pl.BlockSpec((NC, BN, D), lambda i: (0, i, 0)),
            pl.BlockSpec((BN, D), lambda i: (i, 0)),
            pl.BlockSpec((NC, BN, 16), lambda i: (0, i, 0)),
            pl.BlockSpec((1, D), lambda i: (0, 0)),
            pl.BlockSpec((D, D), lambda i: (0, 0)),
        ],
        out_specs=pl.BlockSpec((BN, D), lambda i: (i, 0)),
        out_shape=jax.ShapeDtypeStruct((N, D), jnp.float32),
    )(P1, S1, deg_p, b1, W2)


def _k3_body(p_ref, s2_ref, deg_ref, b2_ref, batch_ref, wfc_ref, bfc_ref,
             out_ref, sums_ref, cnt_ref):
    i = pl.program_id(0)

    @pl.when(i == 0)
    def _():
        sums_ref[...] = jnp.zeros_like(sums_ref)
        cnt_ref[...] = jnp.zeros_like(cnt_ref)

    dinv = _dinv_from_deg(deg_ref)
    agg = p_ref[0] + p_ref[1] + s2_ref[...]
    h2 = dinv[:, None] * agg + b2_ref[...]

    b = batch_ref[0, 0, :]
    gid = lax.broadcasted_iota(jnp.int32, (G, BN), 0)
    M = (gid == b[None, :]).astype(jnp.float32)
    sums_ref[...] += jnp.dot(M, h2, preferred_element_type=jnp.float32)
    cnt_ref[...] += jnp.sum(M, axis=1)[:, None]

    @pl.when(i == NBLK - 1)
    def _():
        mean = sums_ref[...] / jnp.maximum(cnt_ref[...], 1.0)
        out_ref[...] = (jnp.dot(mean, wfc_ref[...],
                                preferred_element_type=jnp.float32)
                        + bfc_ref[...])


def _k3(P2, S2, deg_p, b2, batch3, Wfc, bfc):
    return pl.pallas_call(
        _k3_body,
        grid=(NBLK,),
        in_specs=[
            pl.BlockSpec((NC, BN, D), lambda i: (0, i, 0)),
            pl.BlockSpec((BN, D), lambda i: (i, 0)),
            pl.BlockSpec((NC, BN, 16), lambda i: (0, i, 0)),
            pl.BlockSpec((1, D), lambda i: (0, 0)),
            pl.BlockSpec((1, 1, BN), lambda i: (i, 0, 0)),
            pl.BlockSpec((D, D), lambda i: (0, 0)),
            pl.BlockSpec((1, D), lambda i: (0, 0)),
        ],
        out_specs=pl.BlockSpec((G, D), lambda i: (0, 0)),
        out_shape=jax.ShapeDtypeStruct((G, D), jnp.float32),
        scratch_shapes=[
            pltpu.VMEM((G, D), jnp.float32),
            pltpu.VMEM((G, D), jnp.float32),
        ],
    )(P2, S2, deg_p, b2, batch3, Wfc, bfc)


def kernel(x, edge_index, batch, W1, b1, W2, b2, Wfc, bfc):
    src3 = edge_index[0].reshape(NW, 1, EPW)
    dst4 = edge_index[1].reshape(NW, NCHUNK, 1, CH)
    batch3 = batch.reshape(NBLK, 1, BN)
    ones_rows = jnp.ones((CH, 16), jnp.float32)
    zeros16 = jnp.zeros((RPT, 16), jnp.float32)
    zeros128 = jnp.zeros((RPT, D), jnp.float32)

    deg_p = _deg_kernel(dst4, ones_rows, zeros16)

    S1 = _k1(x, W1, deg_p)
    P1 = _edge_scatter(S1, src3, dst4, zeros128)
    S2 = _k2(P1, S1, deg_p, b1.reshape(1, D), W2)
    P2 = _edge_scatter(S2, src3, dst4, zeros128)
    out = _k3(P2, S2, deg_p, b2.reshape(1, D), batch3, Wfc, bfc.reshape(1, D))
    return (out, batch)
